# baseline (speedup 1.0000x reference)
import jax
import jax.numpy as jnp
from jax.experimental import pallas as pl
from jax.experimental.pallas import tpu as pltpu

MASK_TOKEN_ID = 0
EPS = 1e-6
ROWS_PER_BLOCK = 8
CHUNK = 256
PAD_NEG = -1e30


def _head_kernel(ids_ref, prm_ref, x_ref, o_ref):
    r, v = o_ref.shape
    w = CHUNK
    nfull = v // w
    tail = v - nfull * w

    m = jnp.max(x_ref[:], axis=1, keepdims=True)

    lane = jax.lax.broadcasted_iota(jnp.int32, (r, w), 1)
    neg_inf = jnp.float32(-jnp.inf)

    def step(xc, c, carry):
        v1, v2, v3, i1, i2, i3, z, u = carry
        t = xc - m
        e = jnp.exp(t)
        z = z + e
        u = u + e * t
        b1 = xc > v1
        b2 = xc > v2
        b3 = xc > v3
        v3n = jnp.where(b3, jnp.where(b2, v2, xc), v3)
        i3n = jnp.where(b3, jnp.where(b2, i2, c), i3)
        v2n = jnp.where(b2, jnp.where(b1, v1, xc), v2)
        i2n = jnp.where(b2, jnp.where(b1, i1, c), i2)
        v1n = jnp.where(b1, xc, v1)
        i1n = jnp.where(b1, c, i1)
        return (v1n, v2n, v3n, i1n, i2n, i3n, z, u)

    def body(j, carry):
        xc = x_ref[:, pl.ds(j * w, w)]
        return step(xc, lane + j * w, carry)

    init = (
        jnp.full((r, w), neg_inf), jnp.full((r, w), neg_inf),
        jnp.full((r, w), neg_inf),
        jnp.zeros((r, w), jnp.int32), jnp.zeros((r, w), jnp.int32),
        jnp.zeros((r, w), jnp.int32),
        jnp.zeros((r, w), jnp.float32), jnp.zeros((r, w), jnp.float32),
    )
    carry = jax.lax.fori_loop(0, nfull, body, init)

    if tail:
        xt = x_ref[:, nfull * w:]
        xc = jnp.concatenate(
            [xt, jnp.full((r, w - tail), jnp.float32(PAD_NEG))], axis=1)
        carry = step(xc, lane + nfull * w, carry)

    v1, v2, v3, i1, i2, i3, z, u = carry

    zr = jnp.sum(z, axis=1, keepdims=True)
    sr = jnp.sum(u, axis=1, keepdims=True)
    neg_ent = sr / zr - jnp.log(zr)

    cv = jnp.concatenate([v1, v2, v3], axis=1)
    ci = jnp.concatenate([i1, i2, i3], axis=1)
    big = jnp.int32(1 << 30)
    ik1 = jnp.min(jnp.where(cv == m, ci, big), axis=1, keepdims=True)
    cv = jnp.where(ci == ik1, neg_inf, cv)
    vk2 = jnp.max(cv, axis=1, keepdims=True)
    ik2 = jnp.min(jnp.where(cv == vk2, ci, big), axis=1, keepdims=True)
    cv = jnp.where(ci == ik2, neg_inf, cv)
    vk3 = jnp.max(cv, axis=1, keepdims=True)
    ik3 = jnp.min(jnp.where(cv == vk3, ci, big), axis=1, keepdims=True)

    e2 = jnp.exp(vk2 - m)
    e3 = jnp.exp(vk3 - m)
    tz = 1.0 + e2 + e3

    raw_scale = prm_ref[0, 0]
    raw_centre_neg = prm_ref[0, 1]
    raw_steep = prm_ref[0, 2]
    scale = jax.nn.sigmoid(raw_scale)
    centre = -jax.nn.softplus(raw_centre_neg) - EPS
    steep = jax.nn.softplus(raw_steep) + EPS

    lam = scale * jax.nn.sigmoid(steep * (neg_ent - centre))
    ids = ids_ref[:]
    lam = jnp.where(ids == MASK_TOKEN_ID, lam, 0.0)

    w1 = lam / tz
    w2 = lam * (e2 / tz)
    w3 = lam * (e3 / tz)
    ohv = (1.0 - lam) \
        + jnp.where(ids == ik1, w1, 0.0) \
        + jnp.where(ids == ik2, w2, 0.0) \
        + jnp.where(ids == ik3, w3, 0.0)

    iota = jax.lax.broadcasted_iota(jnp.int32, (r, v), 1)
    o_ref[:] = jnp.where(
        iota == ids, ohv,
        jnp.where(iota == ik1, w1,
                  jnp.where(iota == ik2, w2,
                            jnp.where(iota == ik3, w3, 0.0))))


def kernel(input_ids, logits_prelim, raw_scale, raw_centre_neg, raw_steep, raw_temperature):
    b, s, v = logits_prelim.shape
    n = b * s
    r = ROWS_PER_BLOCK
    x = logits_prelim.reshape(n, v)
    ids = input_ids.reshape(n, 1).astype(jnp.int32)
    prm = jnp.stack(
        [raw_scale, raw_centre_neg, raw_steep, raw_temperature]
    ).reshape(1, 4).astype(jnp.float32)

    out = pl.pallas_call(
        _head_kernel,
        grid=(n // r,),
        in_specs=[
            pl.BlockSpec((r, 1), lambda i: (i, 0)),
            pl.BlockSpec(memory_space=pltpu.SMEM),
            pl.BlockSpec((r, v), lambda i: (i, 0)),
        ],
        out_specs=pl.BlockSpec((r, v), lambda i: (i, 0)),
        out_shape=jax.ShapeDtypeStruct((n, v), jnp.float32),
        compiler_params=pltpu.CompilerParams(
            dimension_semantics=("arbitrary",),
        ),
    )(ids, prm, x)
    return out.reshape(b, s, v)


# speedup vs baseline: 1.9579x; 1.9579x over previous
import jax
import jax.numpy as jnp
from jax.experimental import pallas as pl
from jax.experimental.pallas import tpu as pltpu

MASK_TOKEN_ID = 0
EPS = 1e-6
ROWS_PER_BLOCK = 8
CHUNK = 256
PAD_NEG = -1e30


def _head_kernel(ids_ref, prm_ref, x_ref, o_ref):
    r, v = o_ref.shape
    w = CHUNK
    nfull = v // w
    tail = v - nfull * w

    m = jnp.max(x_ref[:], axis=1, keepdims=True)

    lane = jax.lax.broadcasted_iota(jnp.int32, (r, w), 1)
    neg_inf = jnp.float32(-jnp.inf)

    def step(xc, c, carry):
        v1, v2, v3, i1, i2, i3, z, u = carry
        t = xc - m
        e = jnp.exp(t)
        z = z + e
        u = u + e * t
        b1 = xc > v1
        b2 = xc > v2
        b3 = xc > v3
        v3n = jnp.where(b3, jnp.where(b2, v2, xc), v3)
        i3n = jnp.where(b3, jnp.where(b2, i2, c), i3)
        v2n = jnp.where(b2, jnp.where(b1, v1, xc), v2)
        i2n = jnp.where(b2, jnp.where(b1, i1, c), i2)
        v1n = jnp.where(b1, xc, v1)
        i1n = jnp.where(b1, c, i1)
        return (v1n, v2n, v3n, i1n, i2n, i3n, z, u)

    init = (
        jnp.full((r, w), neg_inf), jnp.full((r, w), neg_inf),
        jnp.full((r, w), neg_inf),
        jnp.zeros((r, w), jnp.int32), jnp.zeros((r, w), jnp.int32),
        jnp.zeros((r, w), jnp.int32),
        jnp.zeros((r, w), jnp.float32), jnp.zeros((r, w), jnp.float32),
    )
    carry = init
    for j in range(nfull):
        carry = step(x_ref[:, j * w:(j + 1) * w], lane + j * w, carry)

    if tail:
        xt = x_ref[:, nfull * w:]
        xc = jnp.concatenate(
            [xt, jnp.full((r, w - tail), jnp.float32(PAD_NEG))], axis=1)
        carry = step(xc, lane + nfull * w, carry)

    v1, v2, v3, i1, i2, i3, z, u = carry

    zr = jnp.sum(z, axis=1, keepdims=True)
    sr = jnp.sum(u, axis=1, keepdims=True)
    neg_ent = sr / zr - jnp.log(zr)

    cv = jnp.concatenate([v1, v2, v3], axis=1)
    ci = jnp.concatenate([i1, i2, i3], axis=1)
    big = jnp.int32(1 << 30)
    ik1 = jnp.min(jnp.where(cv == m, ci, big), axis=1, keepdims=True)
    cv = jnp.where(ci == ik1, neg_inf, cv)
    vk2 = jnp.max(cv, axis=1, keepdims=True)
    ik2 = jnp.min(jnp.where(cv == vk2, ci, big), axis=1, keepdims=True)
    cv = jnp.where(ci == ik2, neg_inf, cv)
    vk3 = jnp.max(cv, axis=1, keepdims=True)
    ik3 = jnp.min(jnp.where(cv == vk3, ci, big), axis=1, keepdims=True)

    e2 = jnp.exp(vk2 - m)
    e3 = jnp.exp(vk3 - m)
    tz = 1.0 + e2 + e3

    raw_scale = prm_ref[0, 0]
    raw_centre_neg = prm_ref[0, 1]
    raw_steep = prm_ref[0, 2]
    scale = jax.nn.sigmoid(raw_scale)
    centre = -jax.nn.softplus(raw_centre_neg) - EPS
    steep = jax.nn.softplus(raw_steep) + EPS

    lam = scale * jax.nn.sigmoid(steep * (neg_ent - centre))
    ids = ids_ref[:]
    lam = jnp.where(ids == MASK_TOKEN_ID, lam, 0.0)

    w1 = lam / tz
    w2 = lam * (e2 / tz)
    w3 = lam * (e3 / tz)
    ohv = (1.0 - lam) \
        + jnp.where(ids == ik1, w1, 0.0) \
        + jnp.where(ids == ik2, w2, 0.0) \
        + jnp.where(ids == ik3, w3, 0.0)

    iota = jax.lax.broadcasted_iota(jnp.int32, (r, v), 1)
    o_ref[:] = jnp.where(
        iota == ids, ohv,
        jnp.where(iota == ik1, w1,
                  jnp.where(iota == ik2, w2,
                            jnp.where(iota == ik3, w3, 0.0))))


def kernel(input_ids, logits_prelim, raw_scale, raw_centre_neg, raw_steep, raw_temperature):
    b, s, v = logits_prelim.shape
    n = b * s
    r = ROWS_PER_BLOCK
    x = logits_prelim.reshape(n, v)
    ids = input_ids.reshape(n, 1).astype(jnp.int32)
    prm = jnp.stack(
        [raw_scale, raw_centre_neg, raw_steep, raw_temperature]
    ).reshape(1, 4).astype(jnp.float32)

    out = pl.pallas_call(
        _head_kernel,
        grid=(n // r,),
        in_specs=[
            pl.BlockSpec((r, 1), lambda i: (i, 0)),
            pl.BlockSpec(memory_space=pltpu.SMEM),
            pl.BlockSpec((r, v), lambda i: (i, 0)),
        ],
        out_specs=pl.BlockSpec((r, v), lambda i: (i, 0)),
        out_shape=jax.ShapeDtypeStruct((n, v), jnp.float32),
        compiler_params=pltpu.CompilerParams(
            dimension_semantics=("arbitrary",),
        ),
    )(ids, prm, x)
    return out.reshape(b, s, v)
